# Initial kernel scaffold; baseline (speedup 1.0000x reference)
import jax
import jax.numpy as jnp
from jax.experimental import pallas as pl


def kernel(emb1, emb2, temperature, W1, b1, W2, b2):
    raise NotImplementedError("write your pallas kernel here")


# speedup vs baseline: 6.2890x; 6.2890x over previous
import jax
import jax.numpy as jnp
from jax.experimental import pallas as pl
from jax.experimental.pallas import tpu as pltpu

_H = 4
_N = 2048
_D = 256
_K = 32
_B = 256


def _body(e1_ref, e2_ref, tau_ref, w1_ref, b1_ref, w2_ref, b2_ref,
          final_ref, adjs_ref):
    adj_heads = []
    for h in range(_H):
        a = e1_ref[h]
        bm = e2_ref[h]
        logits = jnp.dot(a, bm, preferred_element_type=jnp.float32,
                         precision=jax.lax.Precision.DEFAULT)
        s = jnp.maximum(logits, 0.0) / tau_ref[h]
        rowmax = jnp.max(s, axis=1, keepdims=True)
        e = jnp.exp(s - rowmax)
        z = jnp.sum(e, axis=1, keepdims=True)

        def _drop_max(_, x):
            m = jnp.max(x, axis=1, keepdims=True)
            return jnp.where(x >= m, -jnp.inf, x)

        x = jax.lax.fori_loop(0, _K - 1, _drop_max, e)
        t = jnp.max(x, axis=1, keepdims=True)
        masked = jnp.where(e >= t, e, 0.0)
        st = jnp.sum(masked, axis=1, keepdims=True)
        adj_h = masked / (st + 1e-8 * z)
        adjs_ref[h] = adj_h
        adj_heads.append(adj_h)

    adj_mean = (adj_heads[0] + adj_heads[1] + adj_heads[2] + adj_heads[3]) \
        * (1.0 / _H)

    ew = jnp.zeros_like(adj_mean) + b2_ref[0]
    for k in range(2 * _H):
        hk = b1_ref[k]
        acc = adj_heads[0] * w1_ref[k, 0]
        for h in range(1, _H):
            acc = acc + adj_heads[h] * w1_ref[k, h]
        hk = jnp.maximum(acc + hk, 0.0)
        ew = ew + hk * w2_ref[0, k]
    sig = 1.0 / (1.0 + jnp.exp(-ew))
    final_ref[...] = sig * adj_mean


def kernel(emb1, emb2, temperature, W1, b1, W2, b2):
    smem = pl.BlockSpec(memory_space=pltpu.MemorySpace.SMEM)
    grid = (_N // _B,)
    final_adj, adjs = pl.pallas_call(
        _body,
        grid=grid,
        in_specs=[
            pl.BlockSpec((_H, _B, _D), lambda i: (0, i, 0)),
            pl.BlockSpec((_H, _D, _N), lambda i: (0, 0, 0)),
            smem, smem, smem, smem, smem,
        ],
        out_specs=[
            pl.BlockSpec((_B, _N), lambda i: (i, 0)),
            pl.BlockSpec((_H, _B, _N), lambda i: (0, i, 0)),
        ],
        out_shape=[
            jax.ShapeDtypeStruct((_N, _N), jnp.float32),
            jax.ShapeDtypeStruct((_H, _N, _N), jnp.float32),
        ],
    )(emb1, emb2, temperature, W1, b1, W2, b2)
    return (final_adj, adjs)


# speedup vs baseline: 9.8646x; 1.5686x over previous
import jax
import jax.numpy as jnp
from jax.experimental import pallas as pl
from jax.experimental.pallas import tpu as pltpu

_H = 4
_N = 2048
_D = 256
_K = 32
_B = 256


def _body(e1_ref, e2_ref, tau_ref, w1_ref, b1_ref, w2_ref, b2_ref,
          final_ref, adjs_ref):
    adj_heads = []
    for h in range(_H):
        a = e1_ref[h]
        bm = e2_ref[h]
        logits = jnp.dot(a, bm, preferred_element_type=jnp.float32,
                         precision=jax.lax.Precision.DEFAULT)
        s = jnp.maximum(logits, 0.0) / tau_ref[h]
        rowmax = jnp.max(s, axis=1, keepdims=True)
        e = jnp.exp(s - rowmax)
        z = jnp.sum(e, axis=1, keepdims=True)

        def _next_below(_, m):
            return jnp.max(jnp.where(e < m, e, -jnp.inf), axis=1,
                           keepdims=True)

        m0 = jnp.max(e, axis=1, keepdims=True)
        t = jax.lax.fori_loop(0, _K - 1, _next_below, m0)
        masked = jnp.where(e >= t, e, 0.0)
        st = jnp.sum(masked, axis=1, keepdims=True)
        adj_h = masked / (st + 1e-8 * z)
        adjs_ref[h] = adj_h
        adj_heads.append(adj_h)

    adj_mean = (adj_heads[0] + adj_heads[1] + adj_heads[2] + adj_heads[3]) \
        * (1.0 / _H)

    ew = jnp.zeros_like(adj_mean) + b2_ref[0]
    for k in range(2 * _H):
        hk = b1_ref[k]
        acc = adj_heads[0] * w1_ref[k, 0]
        for h in range(1, _H):
            acc = acc + adj_heads[h] * w1_ref[k, h]
        hk = jnp.maximum(acc + hk, 0.0)
        ew = ew + hk * w2_ref[0, k]
    sig = 1.0 / (1.0 + jnp.exp(-ew))
    final_ref[...] = sig * adj_mean


def kernel(emb1, emb2, temperature, W1, b1, W2, b2):
    smem = pl.BlockSpec(memory_space=pltpu.MemorySpace.SMEM)
    grid = (_N // _B,)
    final_adj, adjs = pl.pallas_call(
        _body,
        grid=grid,
        in_specs=[
            pl.BlockSpec((_H, _B, _D), lambda i: (0, i, 0)),
            pl.BlockSpec((_H, _D, _N), lambda i: (0, 0, 0)),
            smem, smem, smem, smem, smem,
        ],
        out_specs=[
            pl.BlockSpec((_B, _N), lambda i: (i, 0)),
            pl.BlockSpec((_H, _B, _N), lambda i: (0, i, 0)),
        ],
        out_shape=[
            jax.ShapeDtypeStruct((_N, _N), jnp.float32),
            jax.ShapeDtypeStruct((_H, _N, _N), jnp.float32),
        ],
    )(emb1, emb2, temperature, W1, b1, W2, b2)
    return (final_adj, adjs)


# speedup vs baseline: 11.3968x; 1.1553x over previous
import jax
import jax.numpy as jnp
from jax.experimental import pallas as pl
from jax.experimental.pallas import tpu as pltpu

_H = 4
_N = 2048
_D = 256
_K = 32
_B = 256


def _body(e1_ref, e2_ref, tau_ref, w1_ref, b1_ref, w2_ref, b2_ref,
          final_ref, adjs_ref):
    es = []
    for h in range(_H):
        a = e1_ref[h]
        bm = e2_ref[h]
        logits = jnp.dot(a, bm, preferred_element_type=jnp.float32,
                         precision=jax.lax.Precision.DEFAULT)
        s = jnp.maximum(logits, 0.0) / tau_ref[h]
        rowmax = jnp.max(s, axis=1, keepdims=True)
        es.append(jnp.exp(s - rowmax))
    e_all = jnp.stack(es, axis=0)
    z = jnp.sum(e_all, axis=2, keepdims=True)

    def _next_below(_, m):
        return jnp.max(jnp.where(e_all < m, e_all, -jnp.inf), axis=2,
                       keepdims=True)

    m0 = jnp.max(e_all, axis=2, keepdims=True)
    t = jax.lax.fori_loop(0, _K - 1, _next_below, m0)
    masked = jnp.where(e_all >= t, e_all, 0.0)
    st = jnp.sum(masked, axis=2, keepdims=True)
    adj = masked / (st + 1e-8 * z)
    adjs_ref[...] = adj
    adj_heads = [adj[h] for h in range(_H)]

    adj_mean = (adj_heads[0] + adj_heads[1] + adj_heads[2] + adj_heads[3]) \
        * (1.0 / _H)

    ew = None
    for k in range(2 * _H):
        acc = adj_heads[0] * w1_ref[k, 0]
        for h in range(1, _H):
            acc = acc + adj_heads[h] * w1_ref[k, h]
        hk = jnp.maximum(acc + b1_ref[k], 0.0)
        contrib = hk * w2_ref[0, k]
        ew = contrib if ew is None else ew + contrib
    sig = 1.0 / (1.0 + jnp.exp(-(ew + b2_ref[0])))
    final_ref[...] = sig * adj_mean


def kernel(emb1, emb2, temperature, W1, b1, W2, b2):
    smem = pl.BlockSpec(memory_space=pltpu.MemorySpace.SMEM)
    grid = (_N // _B,)
    final_adj, adjs = pl.pallas_call(
        _body,
        grid=grid,
        in_specs=[
            pl.BlockSpec((_H, _B, _D), lambda i: (0, i, 0)),
            pl.BlockSpec((_H, _D, _N), lambda i: (0, 0, 0)),
            smem, smem, smem, smem, smem,
        ],
        out_specs=[
            pl.BlockSpec((_B, _N), lambda i: (i, 0)),
            pl.BlockSpec((_H, _B, _N), lambda i: (0, i, 0)),
        ],
        out_shape=[
            jax.ShapeDtypeStruct((_N, _N), jnp.float32),
            jax.ShapeDtypeStruct((_H, _N, _N), jnp.float32),
        ],
    )(emb1, emb2, temperature, W1, b1, W2, b2)
    return (final_adj, adjs)


# speedup vs baseline: 11.8290x; 1.0379x over previous
import jax
import jax.numpy as jnp
from jax.experimental import pallas as pl
from jax.experimental.pallas import tpu as pltpu

_H = 4
_N = 2048
_D = 256
_K = 32
_B = 256


def _body(e1_ref, e2_ref, tau_ref, w1_ref, b1_ref, w2_ref, b2_ref,
          final_ref, adjs_ref):
    es = []
    for h in range(_H):
        a = e1_ref[h]
        bm = e2_ref[h]
        logits = jnp.dot(a, bm, preferred_element_type=jnp.float32,
                         precision=jax.lax.Precision.DEFAULT)
        s = jnp.maximum(logits, 0.0) / tau_ref[h]
        rowmax = jnp.max(s, axis=1, keepdims=True)
        es.append(jnp.exp(s - rowmax))
    e_all = jnp.stack(es, axis=0)
    z = jnp.sum(e_all, axis=2, keepdims=True)

    def _next_below(_, m):
        return jnp.max(jnp.where(e_all < m, e_all, -jnp.inf), axis=2,
                       keepdims=True)

    m0 = jnp.max(e_all, axis=2, keepdims=True)
    t = jax.lax.fori_loop(0, _K - 1, _next_below, m0, unroll=2)
    masked = jnp.where(e_all >= t, e_all, 0.0)
    st = jnp.sum(masked, axis=2, keepdims=True)
    adj = masked / (st + 1e-8 * z)
    adjs_ref[...] = adj
    adj_heads = [adj[h] for h in range(_H)]

    adj_mean = (adj_heads[0] + adj_heads[1] + adj_heads[2] + adj_heads[3]) \
        * (1.0 / _H)

    ew = None
    for k in range(2 * _H):
        acc = adj_heads[0] * w1_ref[k, 0]
        for h in range(1, _H):
            acc = acc + adj_heads[h] * w1_ref[k, h]
        hk = jnp.maximum(acc + b1_ref[k], 0.0)
        contrib = hk * w2_ref[0, k]
        ew = contrib if ew is None else ew + contrib
    sig = 1.0 / (1.0 + jnp.exp(-(ew + b2_ref[0])))
    final_ref[...] = sig * adj_mean


def kernel(emb1, emb2, temperature, W1, b1, W2, b2):
    smem = pl.BlockSpec(memory_space=pltpu.MemorySpace.SMEM)
    grid = (_N // _B,)
    final_adj, adjs = pl.pallas_call(
        _body,
        grid=grid,
        in_specs=[
            pl.BlockSpec((_H, _B, _D), lambda i: (0, i, 0)),
            pl.BlockSpec((_H, _D, _N), lambda i: (0, 0, 0)),
            smem, smem, smem, smem, smem,
        ],
        out_specs=[
            pl.BlockSpec((_B, _N), lambda i: (i, 0)),
            pl.BlockSpec((_H, _B, _N), lambda i: (0, i, 0)),
        ],
        out_shape=[
            jax.ShapeDtypeStruct((_N, _N), jnp.float32),
            jax.ShapeDtypeStruct((_H, _N, _N), jnp.float32),
        ],
    )(emb1, emb2, temperature, W1, b1, W2, b2)
    return (final_adj, adjs)


# speedup vs baseline: 12.0518x; 1.0188x over previous
import jax
import jax.numpy as jnp
from jax.experimental import pallas as pl
from jax.experimental.pallas import tpu as pltpu

_H = 4
_N = 2048
_D = 256
_K = 32
_B = 256


def _body(e1_ref, e2_ref, tau_ref, w1_ref, b1_ref, w2_ref, b2_ref,
          final_ref, adjs_ref):
    es = []
    for h in range(_H):
        a = e1_ref[h]
        bm = e2_ref[h]
        logits = jnp.dot(a, bm, preferred_element_type=jnp.float32,
                         precision=jax.lax.Precision.DEFAULT)
        s = jnp.maximum(logits, 0.0) / tau_ref[h]
        rowmax = jnp.max(s, axis=1, keepdims=True)
        es.append(jnp.exp(s - rowmax))
    e_all = jnp.stack(es, axis=0)
    z = jnp.sum(e_all, axis=2, keepdims=True)

    def _next_below(_, m):
        return jnp.max(jnp.where(e_all < m, e_all, -jnp.inf), axis=2,
                       keepdims=True)

    m0 = jnp.max(e_all, axis=2, keepdims=True)
    t = jax.lax.fori_loop(0, _K - 1, _next_below, m0, unroll=4)
    masked = jnp.where(e_all >= t, e_all, 0.0)
    st = jnp.sum(masked, axis=2, keepdims=True)
    adj = masked / (st + 1e-8 * z)
    adjs_ref[...] = adj
    adj_heads = [adj[h] for h in range(_H)]

    adj_mean = (adj_heads[0] + adj_heads[1] + adj_heads[2] + adj_heads[3]) \
        * (1.0 / _H)

    ew = None
    for k in range(2 * _H):
        acc = adj_heads[0] * w1_ref[k, 0]
        for h in range(1, _H):
            acc = acc + adj_heads[h] * w1_ref[k, h]
        hk = jnp.maximum(acc + b1_ref[k], 0.0)
        contrib = hk * w2_ref[0, k]
        ew = contrib if ew is None else ew + contrib
    sig = 1.0 / (1.0 + jnp.exp(-(ew + b2_ref[0])))
    final_ref[...] = sig * adj_mean


def kernel(emb1, emb2, temperature, W1, b1, W2, b2):
    smem = pl.BlockSpec(memory_space=pltpu.MemorySpace.SMEM)
    grid = (_N // _B,)
    final_adj, adjs = pl.pallas_call(
        _body,
        grid=grid,
        in_specs=[
            pl.BlockSpec((_H, _B, _D), lambda i: (0, i, 0)),
            pl.BlockSpec((_H, _D, _N), lambda i: (0, 0, 0)),
            smem, smem, smem, smem, smem,
        ],
        out_specs=[
            pl.BlockSpec((_B, _N), lambda i: (i, 0)),
            pl.BlockSpec((_H, _B, _N), lambda i: (0, i, 0)),
        ],
        out_shape=[
            jax.ShapeDtypeStruct((_N, _N), jnp.float32),
            jax.ShapeDtypeStruct((_H, _N, _N), jnp.float32),
        ],
    )(emb1, emb2, temperature, W1, b1, W2, b2)
    return (final_adj, adjs)
